# baseline (speedup 1.0000x reference)
import functools

import jax
import jax.numpy as jnp
from jax import lax
from jax.experimental import pallas as pl
from jax.experimental.pallas import tpu as pltpu
from jax.experimental.pallas import tpu_sc as plsc

N = 100000
D = 128
G = 512
NC = 2
NS = 16
NW = NC * NS
L = 16
DJ = D // L
CH = 128
NCHUNKS = -(-N // CH)
TAIL = N - (NCHUNKS - 1) * CH
MAXCH = -(-NCHUNKS // NW)
GPS = G // NS
NBUF = 4
LASTW = (NCHUNKS - 1) // MAXCH
FCAP = 256
FBLK = FCAP // L
SPILL_AT = FCAP - 140


def _sc_body(emb, idxh, zeros, out, rows_v, idx_v, fbuf, fidx, accv, acc,
             gsem):
    c = lax.axis_index("c")
    s = lax.axis_index("s")
    w = c * NS + s
    start = w * MAXCH
    nch = jnp.clip(NCHUNKS - start, 0, MAXCH)
    nfull = nch - jnp.where(w == LASTW, 1, 0)

    pltpu.sync_copy(zeros, rows_v.at[0])
    pltpu.sync_copy(rows_v.at[0, pl.ds(0, GPS)], acc.at[pl.ds(s * GPS, GPS)])
    pltpu.sync_copy(idxh.at[w], idx_v)
    plsc.subcore_barrier()

    @pl.when(w == LASTW)
    def _():
        rb = (NCHUNKS - 1) * CH
        pltpu.sync_copy(emb.at[pl.ds(rb, TAIL)], rows_v.at[0, pl.ds(0, TAIL)])
        pltpu.sync_copy(rows_v.at[0], acc.at[idx_v.at[nch - 1]], add=True)

    def gather(k):
        b = lax.rem(k, NBUF)
        pltpu.async_copy(emb.at[pl.ds((start + k) * CH, CH)], rows_v.at[b],
                         gsem.at[b])

    for k0 in range(NBUF - 1):
        @pl.when(k0 < nfull)
        def _():
            gather(k0)

    zf = jnp.zeros((L,), jnp.float32)
    lanes = lax.iota(jnp.int32, L)

    for j in range(DJ):
        accv[pl.ds(L * j, L)] = zf

    def flush_acc(prev, count):
        for j in range(DJ):
            fbuf[count, pl.ds(L * j, L)] = accv[pl.ds(L * j, L)]
            accv[pl.ds(L * j, L)] = zf
        row = count >> 4
        rv = fidx[row, pl.ds(0, L)]
        fidx[row, pl.ds(0, L)] = jnp.where(lanes == (count & (L - 1)), prev,
                                           rv)

    def spill(count):
        nblk = (count + L - 1) >> 4
        rem = count & (L - 1)

        @pl.when(rem != 0)
        def _():
            rv = fidx[nblk - 1, pl.ds(0, L)]
            fidx[nblk - 1, pl.ds(0, L)] = jnp.where(lanes < rem, rv, 0)

        def zpad(i, carry):
            for j in range(DJ):
                fbuf[i, pl.ds(L * j, L)] = zf
            return carry

        lax.fori_loop(count, nblk * L, zpad, 0)
        for t in range(FBLK):
            @pl.when(t < nblk)
            def _():
                pltpu.sync_copy(fbuf.at[pl.ds(t * L, L)], acc.at[fidx.at[t]],
                                add=True)

    def proc_group(k, b, g, prev, count):
        iv = idx_v[k, pl.ds(g * L, L)]
        mn = iv[0]
        mx = iv[L - 1]
        uniform = mx == mn
        hit = mn == prev
        trig = jnp.logical_not(hit)

        @pl.when(trig)
        def _():
            flush_acc(prev, count)

        count2 = count + trig.astype(jnp.int32)

        @pl.when(uniform)
        def _():
            for j in range(DJ):
                v = accv[pl.ds(L * j, L)]
                for r in range(L):
                    v = v + rows_v[b, g * L + r, pl.ds(L * j, L)]
                accv[pl.ds(L * j, L)] = v

        @pl.when(jnp.logical_not(uniform))
        def _():
            off = count2 & (L - 1)
            rot = iv[(lanes - off) & (L - 1)]
            row0 = count2 >> 4
            rv = fidx[row0, pl.ds(0, L)]
            fidx[row0, pl.ds(0, L)] = jnp.where(lanes >= off, rot, rv)

            @pl.when(off != 0)
            def _():
                rv2 = fidx[row0 + 1, pl.ds(0, L)]
                fidx[row0 + 1, pl.ds(0, L)] = jnp.where(lanes < off, rot, rv2)

            for j in range(DJ):
                v0 = rows_v[b, g * L, pl.ds(L * j, L)]

                @pl.when(hit)
                def _():
                    fbuf[count2, pl.ds(L * j, L)] = v0 + accv[pl.ds(L * j, L)]
                    accv[pl.ds(L * j, L)] = zf

                @pl.when(jnp.logical_not(hit))
                def _():
                    fbuf[count2, pl.ds(L * j, L)] = v0

            for r in range(1, L):
                for j in range(DJ):
                    fbuf[count2 + r, pl.ds(L * j, L)] = (
                        rows_v[b, g * L + r, pl.ds(L * j, L)])

        count3 = count2 + jnp.where(uniform, 0, L)
        return mx, count3

    def step(k, carry):
        prev, count = carry
        b = lax.rem(k, NBUF)

        @pl.when(k + (NBUF - 1) < nfull)
        def _():
            gather(k + (NBUF - 1))

        pltpu.make_async_copy(emb.at[pl.ds(0, CH)], rows_v.at[b],
                              gsem.at[b]).wait()

        def gbody(g, c2):
            return proc_group(k, b, g, c2[0], c2[1])

        prev, count = lax.fori_loop(0, CH // L, gbody, (prev, count))
        spl = count >= SPILL_AT

        @pl.when(spl)
        def _():
            spill(count)

        return prev, jnp.where(spl, 0, count)

    prev, count = lax.fori_loop(0, nfull, step,
                                (jnp.int32(0), jnp.int32(0)))
    flush_acc(prev, count)
    spill(count + 1)

    plsc.subcore_barrier()
    pltpu.sync_copy(acc.at[pl.ds(s * GPS, GPS)], out.at[c, pl.ds(s * GPS, GPS)])


_sc_segsum = functools.partial(
    pl.kernel,
    out_type=jax.ShapeDtypeStruct((NC, G, D), jnp.float32),
    mesh=plsc.VectorSubcoreMesh(core_axis_name="c", subcore_axis_name="s"),
    name="sc_segment_sum",
    scratch_types=[
        pltpu.VMEM((NBUF, CH, D), jnp.float32),
        pltpu.VMEM((MAXCH, CH), jnp.int32),
        pltpu.VMEM((FCAP, D), jnp.float32),
        pltpu.VMEM((FBLK, L), jnp.int32),
        pltpu.VMEM((D,), jnp.float32),
        pltpu.VMEM_SHARED((G, D), jnp.float32),
        pltpu.SemaphoreType.DMA((NBUF,)),
    ],
)(_sc_body)


def _mlp_body(p_ref, w1_ref, b1_ref, w2_ref, b2_ref, o_ref):
    g = p_ref[0] + p_ref[1]
    h = lax.dot_general(g, w1_ref[...], (((1,), (1,)), ((), ())),
                        preferred_element_type=jnp.float32)
    h = jnp.maximum(h + b1_ref[...], 0.0)
    o_ref[...] = lax.dot_general(h, w2_ref[...], (((1,), (1,)), ((), ())),
                                 preferred_element_type=jnp.float32) + b2_ref[...]


_tc_mlp = pl.pallas_call(
    _mlp_body,
    out_shape=jax.ShapeDtypeStruct((G, D), jnp.float32),
)


def kernel(node_embeddings, batch_indices, W1, b1, W2, b2):
    idx = batch_indices.astype(jnp.int32)
    idx3 = jnp.pad(idx, (0, NW * MAXCH * CH - N)).reshape(NW, MAXCH, CH)
    zeros = jnp.zeros((CH, D), jnp.float32)
    partials = _sc_segsum(node_embeddings, idx3, zeros)
    return _tc_mlp(partials, W1, b1.reshape(1, D), W2, b2.reshape(1, D))


# speedup vs baseline: 1.5194x; 1.5194x over previous
import functools

import jax
import jax.numpy as jnp
from jax import lax
from jax.experimental import pallas as pl
from jax.experimental.pallas import tpu as pltpu
from jax.experimental.pallas import tpu_sc as plsc

N = 100000
D = 128
G = 512
NC = 2
NS = 16
NW = NC * NS
CH = 128
NCHUNKS = -(-N // CH)
TAIL = N - (NCHUNKS - 1) * CH
MAXCH = -(-NCHUNKS // NW)
GPS = G // NS
NBUF = 4
LASTW = (NCHUNKS - 1) // MAXCH


def _sc_body(emb, idxh, zeros, out, rows_v, idx_v, acc, gsem, ssem):
    c = lax.axis_index("c")
    s = lax.axis_index("s")
    w = c * NS + s
    start = w * MAXCH
    nch = jnp.clip(NCHUNKS - start, 0, MAXCH)
    nfull = nch - jnp.where(w == LASTW, 1, 0)

    pltpu.sync_copy(zeros, rows_v.at[0])
    pltpu.sync_copy(rows_v.at[0, pl.ds(0, GPS)], acc.at[pl.ds(s * GPS, GPS)])
    pltpu.sync_copy(idxh.at[w], idx_v)
    plsc.subcore_barrier()

    @pl.when(w == LASTW)
    def _():
        rb = (NCHUNKS - 1) * CH
        pltpu.sync_copy(emb.at[pl.ds(rb, TAIL)], rows_v.at[0, pl.ds(0, TAIL)])
        pltpu.sync_copy(rows_v.at[0], acc.at[idx_v.at[nch - 1]], add=True)

    def gather(k):
        b = lax.rem(k, NBUF)
        pltpu.async_copy(emb.at[pl.ds((start + k) * CH, CH)], rows_v.at[b],
                         gsem.at[b])

    def wait_scatter(b):
        pltpu.make_async_copy(rows_v.at[b], acc.at[idx_v.at[0]],
                              ssem.at[b]).wait()

    for k0 in range(NBUF - 1):
        @pl.when(k0 < nfull)
        def _():
            gather(k0)

    def step(k, carry):
        b = lax.rem(k, NBUF)

        @pl.when(k + (NBUF - 1) < nfull)
        def _():
            @pl.when(k >= 1)
            def _():
                wait_scatter(lax.rem(k + NBUF - 1, NBUF))
            gather(k + (NBUF - 1))

        pltpu.make_async_copy(emb.at[pl.ds(0, CH)], rows_v.at[b],
                              gsem.at[b]).wait()
        pltpu.async_copy(rows_v.at[b], acc.at[idx_v.at[k]], ssem.at[b],
                         add=True)
        return carry

    lax.fori_loop(0, nfull, step, 0)

    def drain(j, carry):
        wait_scatter(lax.rem(j, NBUF))
        return carry

    lax.fori_loop(jnp.maximum(nfull - NBUF, 0), nfull, drain, 0)
    plsc.subcore_barrier()
    pltpu.sync_copy(acc.at[pl.ds(s * GPS, GPS)], out.at[c, pl.ds(s * GPS, GPS)])


_sc_segsum = functools.partial(
    pl.kernel,
    out_type=jax.ShapeDtypeStruct((NC, G, D), jnp.float32),
    mesh=plsc.VectorSubcoreMesh(core_axis_name="c", subcore_axis_name="s"),
    name="sc_segment_sum",
    scratch_types=[
        pltpu.VMEM((NBUF, CH, D), jnp.float32),
        pltpu.VMEM((MAXCH, CH), jnp.int32),
        pltpu.VMEM_SHARED((G, D), jnp.float32),
        pltpu.SemaphoreType.DMA((NBUF,)),
        pltpu.SemaphoreType.DMA((NBUF,)),
    ],
)(_sc_body)


def _mlp_body(p_ref, w1_ref, b1_ref, w2_ref, b2_ref, o_ref):
    g = p_ref[0] + p_ref[1]
    h = lax.dot_general(g, w1_ref[...], (((1,), (1,)), ((), ())),
                        preferred_element_type=jnp.float32)
    h = jnp.maximum(h + b1_ref[...], 0.0)
    o_ref[...] = lax.dot_general(h, w2_ref[...], (((1,), (1,)), ((), ())),
                                 preferred_element_type=jnp.float32) + b2_ref[...]


_tc_mlp = pl.pallas_call(
    _mlp_body,
    out_shape=jax.ShapeDtypeStruct((G, D), jnp.float32),
)


def kernel(node_embeddings, batch_indices, W1, b1, W2, b2):
    idx = batch_indices.astype(jnp.int32)
    idx3 = jnp.pad(idx, (0, NW * MAXCH * CH - N)).reshape(NW, MAXCH, CH)
    zeros = jnp.zeros((CH, D), jnp.float32)
    partials = _sc_segsum(node_embeddings, idx3, zeros)
    return _tc_mlp(partials, W1, b1.reshape(1, D), W2, b2.reshape(1, D))
